# baseline (speedup 1.0000x reference)
import functools

import jax
import jax.numpy as jnp
from jax import lax
from jax.experimental import pallas as pl
from jax.experimental.pallas import tpu as pltpu
from jax.experimental.pallas import tpu_sc as plsc


def _sc_gather(idx, table):
    B = idx.shape[0]
    D = table.shape[1]
    try:
        info = plsc.get_sparse_core_info()
        nc, ns = info.num_cores, info.num_subcores
    except Exception:
        nc, ns = 2, 16
    nw = nc * ns
    b_per_w = B // nw
    mesh = plsc.VectorSubcoreMesh(core_axis_name="c", subcore_axis_name="s")

    @functools.partial(
        pl.kernel,
        mesh=mesh,
        out_type=jax.ShapeDtypeStruct((B, D), jnp.float32),
        scratch_types=[
            pltpu.VMEM((b_per_w,), jnp.int32),
            pltpu.VMEM((b_per_w, D), jnp.float32),
            pltpu.SemaphoreType.DMA,
        ],
        compiler_params=pltpu.CompilerParams(use_tc_tiling_on_sc=False),
    )
    def gather_kernel(idx_hbm, table_hbm, out_hbm, idx_v, rows_v, sem):
        wid = lax.axis_index("s") * nc + lax.axis_index("c")
        base = wid * b_per_w
        pltpu.sync_copy(idx_hbm.at[pl.ds(base, b_per_w)], idx_v)
        pltpu.async_copy(table_hbm.at[idx_v], rows_v, sem).wait()
        pltpu.sync_copy(rows_v, out_hbm.at[pl.ds(base, b_per_w)])

    return gather_kernel(idx, table)


def _tc_project_kernel(u_ref, ud_ref, gp_ref, ap_ref, op_ref, wt_ref, b_ref,
                       out_ref, *, blk):
    wt = wt_ref[:]
    acc = jnp.dot(u_ref[:], wt[0:64], preferred_element_type=jnp.float32)
    c = jnp.concatenate([
        jnp.dot(gp_ref[:], wt[64:128], preferred_element_type=jnp.float32),
        jnp.dot(ap_ref[:], wt[128:192], preferred_element_type=jnp.float32),
        jnp.dot(op_ref[:], wt[192:256], preferred_element_type=jnp.float32),
    ], axis=0)
    ud = ud_ref[:]
    g = ud[:, 1:2]
    a = ud[:, 2:3]
    o = ud[:, 3:4]
    col = lax.broadcasted_iota(jnp.int32, (blk, 40), 1)
    m = ((col == g) | (col == 8 + a) | (col == 16 + o)).astype(jnp.float32)
    out_ref[:] = acc + jnp.dot(m, c, preferred_element_type=jnp.float32) \
        + b_ref[:]


def _tc_project(u_e, user_data, gp, ap, op, wt, b2d, *, blk=2048,
                interpret=False):
    B, D = u_e.shape
    grid = (B // blk,)
    return pl.pallas_call(
        functools.partial(_tc_project_kernel, blk=blk),
        grid=grid,
        in_specs=[
            pl.BlockSpec((blk, D), lambda i: (i, 0)),
            pl.BlockSpec((blk, 4), lambda i: (i, 0)),
            pl.BlockSpec(gp.shape, lambda i: (0, 0)),
            pl.BlockSpec(ap.shape, lambda i: (0, 0)),
            pl.BlockSpec(op.shape, lambda i: (0, 0)),
            pl.BlockSpec(wt.shape, lambda i: (0, 0)),
            pl.BlockSpec(b2d.shape, lambda i: (0, 0)),
        ],
        out_specs=pl.BlockSpec((blk, D), lambda i: (i, 0)),
        out_shape=jax.ShapeDtypeStruct((B, D), jnp.float32),
        interpret=interpret,
    )(u_e, user_data, gp, ap, op, wt, b2d)


def kernel(user_data, user_table, gender_table, age_table, occupation_table,
           W, b):
    idx = user_data[:, 0].astype(jnp.int32)
    u_e = _sc_gather(idx, user_table)
    gp = jnp.zeros((8, 64), jnp.float32).at[:3].set(gender_table)
    ap = age_table
    op = jnp.zeros((24, 64), jnp.float32).at[:22].set(occupation_table)
    wt = W.T
    b2d = b.reshape(1, 64)
    return _tc_project(u_e, user_data.astype(jnp.int32), gp, ap, op, wt, b2d)
